# baseline (speedup 1.0000x reference)
import functools

import jax
import jax.numpy as jnp
from jax import lax
from jax.experimental import pallas as pl
from jax.experimental.pallas import tpu as pltpu
from jax.experimental.pallas import tpu_sc as plsc

N = 10000
H = 128
E = 320000
NC = 2
NS = 16
NW = NC * NS
EPW = E // NW
CB = 80
NCHUNK = EPW // CB
NBLK = 5
CPB = NCHUNK // NBLK
NPAD = 10240
RPW = NPAD // NS
DEGW = 128
ZR = 128

_mesh = plsc.VectorSubcoreMesh(core_axis_name="c", subcore_axis_name="s")


@functools.partial(
    pl.kernel,
    mesh=_mesh,
    out_type=jax.ShapeDtypeStruct((NC * NPAD,), jnp.float32),
    scratch_types=[
        pltpu.VMEM((CPB, CB), jnp.int32),
        pltpu.VMEM((CB, DEGW), jnp.float32),
        pltpu.VMEM((ZR, DEGW), jnp.float32),
        pltpu.VMEM((RPW,), jnp.float32),
        pltpu.VMEM_SHARED((NPAD, DEGW), jnp.float32),
        pltpu.SemaphoreType.DMA,
    ],
)
def _deg_kernel(dst4, ones_hbm, zeros_hbm, deg_out,
                idx_v, ones_v, nb_v, cp_v, acc, sem):
    c = lax.axis_index("c")
    s = lax.axis_index("s")
    w = c * NS + s
    pltpu.sync_copy(ones_hbm, ones_v)
    for r in range(RPW // ZR):
        pltpu.sync_copy(zeros_hbm, acc.at[pl.ds(s * RPW + r * ZR, ZR)])
    plsc.subcore_barrier()

    def blk_body(blk, carry):
        pltpu.sync_copy(dst4.at[w, blk], idx_v)

        def body(j, carry2):
            pltpu.async_copy(ones_v, acc.at[idx_v.at[j]], sem, add=True)
            return carry2

        lax.fori_loop(0, CPB, body, 0)

        def drain(j, carry2):
            pltpu.make_async_copy(ones_v, acc.at[idx_v.at[0]], sem).wait()
            return carry2

        lax.fori_loop(0, CPB, drain, 0)
        return carry

    lax.fori_loop(0, NBLK, blk_body, 0)
    plsc.subcore_barrier()

    lanes = lax.iota(jnp.int32, 16)
    for half in range(RPW // ZR):
        pltpu.sync_copy(acc.at[pl.ds(s * RPW + half * ZR, ZR)], nb_v)
        for g in range(ZR // 16):
            acc16 = jnp.zeros((16,), jnp.float32)
            for j in range(16):
                v = nb_v[16 * g + j, pl.ds(0, 16)]
                mask = jnp.where(lanes == j, 1.0, 0.0).astype(jnp.float32)
                acc16 = acc16 + v * mask
            cp_v[pl.ds(half * ZR + 16 * g, 16)] = acc16
    for kb in range(RPW // 128):
        pltpu.sync_copy(
            cp_v.at[pl.ds(kb * 128, 128)],
            deg_out.at[pl.ds((s * (RPW // 128) + kb) * 256 + c * 128, 128)],
        )


@functools.partial(
    pl.kernel,
    mesh=_mesh,
    out_type=jax.ShapeDtypeStruct((NC * NPAD, H), jnp.float32),
    scratch_types=[
        pltpu.VMEM((CPB, CB), jnp.int32),
        pltpu.VMEM((CPB, CB), jnp.int32),
        pltpu.VMEM((CB, H), jnp.float32),
        pltpu.VMEM((CB, H), jnp.float32),
        pltpu.VMEM((CB, H), jnp.float32),
        pltpu.VMEM((CB, H), jnp.float32),
        pltpu.VMEM_SHARED((NPAD, H), jnp.float32),
        pltpu.SemaphoreType.DMA,
        pltpu.SemaphoreType.DMA,
        pltpu.SemaphoreType.DMA,
        pltpu.SemaphoreType.DMA,
        pltpu.SemaphoreType.DMA,
        pltpu.SemaphoreType.DMA,
        pltpu.SemaphoreType.DMA,
        pltpu.SemaphoreType.DMA,
    ],
)
def _agg_kernel(h2_hbm, src4, dst4, zeros_hbm, agg_out,
                src_v, dst_v, rows_0, rows_1, rows_2, rows_3, acc,
                gs0, gs1, gs2, gs3, ss0, ss1, ss2, ss3):
    c = lax.axis_index("c")
    s = lax.axis_index("s")
    w = c * NS + s
    rows = [rows_0, rows_1, rows_2, rows_3]
    gs = [gs0, gs1, gs2, gs3]
    ss = [ss0, ss1, ss2, ss3]
    for r in range(RPW // ZR):
        pltpu.sync_copy(zeros_hbm, acc.at[pl.ds(s * RPW + r * ZR, ZR)])
    plsc.subcore_barrier()

    def gath(j, b):
        pltpu.async_copy(h2_hbm.at[src_v.at[j]], rows[b], gs[b])

    def gwait(j, b):
        pltpu.make_async_copy(h2_hbm.at[src_v.at[j]], rows[b], gs[b]).wait()

    def scat(j, b):
        pltpu.async_copy(rows[b], acc.at[dst_v.at[j]], ss[b], add=True)

    def swait(j, b):
        pltpu.make_async_copy(rows[b], acc.at[dst_v.at[j]], ss[b]).wait()

    def blk_body(blk, carry):
        pltpu.sync_copy(src4.at[w, blk], src_v)
        pltpu.sync_copy(dst4.at[w, blk], dst_v)
        gath(0, 0)
        gath(1, 1)
        gath(2, 2)
        for j in range(CPB):
            b = j % 4
            gwait(j, b)
            scat(j, b)
            if j + 3 < CPB:
                if j >= 1:
                    swait(j - 1, (j + 3) % 4)
                gath(j + 3, (j + 3) % 4)
        for j in range(CPB - 4, CPB):
            swait(j, j % 4)
        return carry

    lax.fori_loop(0, NBLK, blk_body, 0)
    plsc.subcore_barrier()
    pltpu.sync_copy(
        acc.at[pl.ds(s * RPW, RPW)], agg_out.at[pl.ds(c * NPAD + s * RPW, RPW)]
    )


_RB = 128
_NBK = 79


def _dinv_col(degc_ref):
    dd = degc_ref[0]
    d = dd[:, 0:128] + dd[:, 128:256] + 1.0
    dinv = lax.rsqrt(d)
    return jnp.transpose(jnp.broadcast_to(dinv, (_RB, _RB)))


def _mm_body(x_ref, w_ref, degc_ref, h2_ref):
    h2_ref[...] = (
        jnp.dot(x_ref[...], w_ref[...], preferred_element_type=jnp.float32)
        * _dinv_col(degc_ref)
    )


_mm_call = pl.pallas_call(
    _mm_body,
    grid=(_NBK,),
    in_specs=[
        pl.BlockSpec((_RB, H), lambda i: (i, 0)),
        pl.BlockSpec((H, H), lambda i: (0, 0)),
        pl.BlockSpec((1, 1, 2 * _RB), lambda i: (i, 0, 0)),
    ],
    out_specs=pl.BlockSpec((_RB, H), lambda i: (i, 0)),
    out_shape=jax.ShapeDtypeStruct((N, H), jnp.float32),
)


def _fin_body(aggp_ref, h2_ref, degc_ref, b_ref, g_ref, be_ref, out_ref):
    pre = _dinv_col(degc_ref) * (aggp_ref[0] + aggp_ref[1] + h2_ref[...]) \
        + b_ref[...]
    ge = 0.5 * pre * (1.0 + lax.erf(pre * 0.7071067811865476))
    mean = jnp.mean(ge, axis=-1, keepdims=True)
    cent = ge - mean
    var = jnp.mean(cent * cent, axis=-1, keepdims=True)
    out_ref[...] = cent * lax.rsqrt(var + 1e-5) * g_ref[...] + be_ref[...]


_fin_call = pl.pallas_call(
    _fin_body,
    grid=(_NBK,),
    in_specs=[
        pl.BlockSpec((NC, _RB, H), lambda i: (0, i, 0)),
        pl.BlockSpec((_RB, H), lambda i: (i, 0)),
        pl.BlockSpec((1, 1, 2 * _RB), lambda i: (i, 0, 0)),
        pl.BlockSpec((1, H), lambda i: (0, 0)),
        pl.BlockSpec((1, H), lambda i: (0, 0)),
        pl.BlockSpec((1, H), lambda i: (0, 0)),
    ],
    out_specs=pl.BlockSpec((_RB, H), lambda i: (i, 0)),
    out_shape=jax.ShapeDtypeStruct((N, H), jnp.float32),
)


def kernel(x, edge_index, W, b, gamma, beta):
    src3 = edge_index[0].astype(jnp.int32).reshape(NW, NCHUNK, CB)
    dst3 = edge_index[1].astype(jnp.int32).reshape(NW, NCHUNK, CB)
    src4 = src3.reshape(NW, NBLK, CPB, CB)
    dst4 = dst3.reshape(NW, NBLK, CPB, CB)
    zeros_blk = jnp.zeros((ZR, H), jnp.float32)
    ones_deg = jnp.ones((CB, DEGW), jnp.float32)

    degc = _deg_kernel(dst4, ones_deg, zeros_blk).reshape(NPAD // 128, 1, 256)
    h2 = _mm_call(x, W, degc)
    aggp = _agg_kernel(h2, src4, dst4, zeros_blk).reshape(NC, NPAD, H)
    return _fin_call(aggp, h2, degc, b.reshape(1, H), gamma.reshape(1, H),
                     beta.reshape(1, H))


# speedup vs baseline: 1.2043x; 1.2043x over previous
import functools

import jax
import jax.numpy as jnp
from jax import lax
from jax.experimental import pallas as pl
from jax.experimental.pallas import tpu as pltpu
from jax.experimental.pallas import tpu_sc as plsc

N = 10000
H = 128
E = 320000
NC = 2
NS = 16
NW = NC * NS
EPW = E // NW
CB = 80
NCHUNK = EPW // CB
NBLK = 5
CPB = NCHUNK // NBLK
NPAD = 10240
RPW = NPAD // NS
DEGW = 128
ZR = 128

_mesh = plsc.VectorSubcoreMesh(core_axis_name="c", subcore_axis_name="s")


@functools.partial(
    pl.kernel,
    mesh=_mesh,
    out_type=jax.ShapeDtypeStruct((NC * NPAD, DEGW), jnp.float32),
    scratch_types=[
        pltpu.VMEM((NCHUNK, CB), jnp.int32),
        pltpu.VMEM((CB, DEGW), jnp.float32),
        pltpu.VMEM_SHARED((NPAD, DEGW), jnp.float32),
        pltpu.SemaphoreType.DMA,
    ],
)
def _deg_kernel(dst3, ones_hbm, zeros_hbm, deg_out, idx_v, ones_v, acc, sem):
    c = lax.axis_index("c")
    s = lax.axis_index("s")
    w = c * NS + s
    pltpu.sync_copy(dst3.at[w], idx_v)
    pltpu.sync_copy(ones_hbm, ones_v)
    for r in range(RPW // ZR):
        pltpu.sync_copy(zeros_hbm, acc.at[pl.ds(s * RPW + r * ZR, ZR)])
    plsc.subcore_barrier()

    def body(j, carry):
        pltpu.async_copy(ones_v, acc.at[idx_v.at[j]], sem, add=True)
        return carry

    lax.fori_loop(0, NCHUNK, body, 0)

    def drain(j, carry):
        pltpu.make_async_copy(ones_v, acc.at[idx_v.at[0]], sem).wait()
        return carry

    lax.fori_loop(0, NCHUNK, drain, 0)
    plsc.subcore_barrier()
    pltpu.sync_copy(
        acc.at[pl.ds(s * RPW, RPW)], deg_out.at[pl.ds(c * NPAD + s * RPW, RPW)]
    )


@functools.partial(
    pl.kernel,
    mesh=_mesh,
    out_type=jax.ShapeDtypeStruct((NC * NPAD, H), jnp.float32),
    scratch_types=[
        pltpu.VMEM((CPB, CB), jnp.int32),
        pltpu.VMEM((CPB, CB), jnp.int32),
        pltpu.VMEM((CB, H), jnp.float32),
        pltpu.VMEM((CB, H), jnp.float32),
        pltpu.VMEM((CB, H), jnp.float32),
        pltpu.VMEM((CB, H), jnp.float32),
        pltpu.VMEM_SHARED((NPAD, H), jnp.float32),
        pltpu.SemaphoreType.DMA,
        pltpu.SemaphoreType.DMA,
        pltpu.SemaphoreType.DMA,
        pltpu.SemaphoreType.DMA,
        pltpu.SemaphoreType.DMA,
        pltpu.SemaphoreType.DMA,
        pltpu.SemaphoreType.DMA,
        pltpu.SemaphoreType.DMA,
    ],
)
def _agg_kernel(h2_hbm, src4, dst4, zeros_hbm, agg_out,
                src_v, dst_v, rows_0, rows_1, rows_2, rows_3, acc,
                gs0, gs1, gs2, gs3, ss0, ss1, ss2, ss3):
    c = lax.axis_index("c")
    s = lax.axis_index("s")
    w = c * NS + s
    rows = [rows_0, rows_1, rows_2, rows_3]
    gs = [gs0, gs1, gs2, gs3]
    ss = [ss0, ss1, ss2, ss3]
    for r in range(RPW // ZR):
        pltpu.sync_copy(zeros_hbm, acc.at[pl.ds(s * RPW + r * ZR, ZR)])
    plsc.subcore_barrier()

    def gath(j, b):
        pltpu.async_copy(h2_hbm.at[src_v.at[j]], rows[b], gs[b])

    def gwait(j, b):
        pltpu.make_async_copy(h2_hbm.at[src_v.at[j]], rows[b], gs[b]).wait()

    def scat(j, b):
        pltpu.async_copy(rows[b], acc.at[dst_v.at[j]], ss[b], add=True)

    def swait(j, b):
        pltpu.make_async_copy(rows[b], acc.at[dst_v.at[j]], ss[b]).wait()

    def blk_body(blk, carry):
        pltpu.sync_copy(src4.at[w, blk], src_v)
        pltpu.sync_copy(dst4.at[w, blk], dst_v)
        gath(0, 0)
        gath(1, 1)
        gath(2, 2)
        for j in range(CPB):
            b = j % 4
            gwait(j, b)
            scat(j, b)
            if j + 3 < CPB:
                if j >= 1:
                    swait(j - 1, (j + 3) % 4)
                gath(j + 3, (j + 3) % 4)
        for j in range(CPB - 4, CPB):
            swait(j, j % 4)
        return carry

    lax.fori_loop(0, NBLK, blk_body, 0)
    plsc.subcore_barrier()
    pltpu.sync_copy(
        acc.at[pl.ds(s * RPW, RPW)], agg_out.at[pl.ds(c * NPAD + s * RPW, RPW)]
    )


_RB = 1000
_NBK = N // _RB


def _dinv_col(degp_ref):
    return lax.rsqrt(degp_ref[0, :, 0:1] + degp_ref[1, :, 0:1] + 1.0)


def _mm_body(x_ref, w_ref, degp_ref, h2_ref):
    h2_ref[...] = (
        jnp.dot(x_ref[...], w_ref[...], preferred_element_type=jnp.float32)
        * _dinv_col(degp_ref)
    )


_mm_call = pl.pallas_call(
    _mm_body,
    grid=(_NBK,),
    in_specs=[
        pl.BlockSpec((_RB, H), lambda i: (i, 0)),
        pl.BlockSpec((H, H), lambda i: (0, 0)),
        pl.BlockSpec((NC, _RB, DEGW), lambda i: (0, i, 0)),
    ],
    out_specs=pl.BlockSpec((_RB, H), lambda i: (i, 0)),
    out_shape=jax.ShapeDtypeStruct((N, H), jnp.float32),
)


def _fin_body(aggp_ref, h2_ref, degp_ref, b_ref, g_ref, be_ref, out_ref):
    pre = _dinv_col(degp_ref) * (aggp_ref[0] + aggp_ref[1] + h2_ref[...]) \
        + b_ref[...]
    ge = 0.5 * pre * (1.0 + lax.erf(pre * 0.7071067811865476))
    mean = jnp.mean(ge, axis=-1, keepdims=True)
    cent = ge - mean
    var = jnp.mean(cent * cent, axis=-1, keepdims=True)
    out_ref[...] = cent * lax.rsqrt(var + 1e-5) * g_ref[...] + be_ref[...]


_fin_call = pl.pallas_call(
    _fin_body,
    grid=(_NBK,),
    in_specs=[
        pl.BlockSpec((NC, _RB, H), lambda i: (0, i, 0)),
        pl.BlockSpec((_RB, H), lambda i: (i, 0)),
        pl.BlockSpec((NC, _RB, DEGW), lambda i: (0, i, 0)),
        pl.BlockSpec((1, H), lambda i: (0, 0)),
        pl.BlockSpec((1, H), lambda i: (0, 0)),
        pl.BlockSpec((1, H), lambda i: (0, 0)),
    ],
    out_specs=pl.BlockSpec((_RB, H), lambda i: (i, 0)),
    out_shape=jax.ShapeDtypeStruct((N, H), jnp.float32),
)


def kernel(x, edge_index, W, b, gamma, beta):
    src3 = edge_index[0].astype(jnp.int32).reshape(NW, NCHUNK, CB)
    dst3 = edge_index[1].astype(jnp.int32).reshape(NW, NCHUNK, CB)
    src4 = src3.reshape(NW, NBLK, CPB, CB)
    dst4 = dst3.reshape(NW, NBLK, CPB, CB)
    zeros_blk = jnp.zeros((ZR, H), jnp.float32)
    ones_deg = jnp.ones((CB, DEGW), jnp.float32)

    degp = _deg_kernel(dst3, ones_deg, zeros_blk).reshape(NC, NPAD, DEGW)
    h2 = _mm_call(x, W, degp)
    aggp = _agg_kernel(h2, src4, dst4, zeros_blk).reshape(NC, NPAD, H)
    return _fin_call(aggp, h2, degp, b.reshape(1, H), gamma.reshape(1, H),
                     beta.reshape(1, H))
